# baseline (device time: 11526 ns/iter reference)
import jax
import jax.numpy as jnp
from jax import lax
from jax.experimental import pallas as pl
from jax.experimental.pallas import tpu as pltpu

N_DEV = 8
N_BLK = 4
SHIFT = 16.0


def kernel(x):
    m, n = x.shape
    mb = m // N_BLK

    def body(x_ref, out_ref, mine_ref, comm_ref, send_sems, recv_sems):
        my_pos = lax.axis_index("i")

        barrier_sem = pltpu.get_barrier_semaphore()
        for p in range(N_DEV):
            @pl.when(my_pos != p)
            def _():
                pl.semaphore_signal(
                    barrier_sem, inc=1,
                    device_id=(p,), device_id_type=pl.DeviceIdType.MESH,
                )

        for b in range(N_BLK):
            xv = x_ref[b * mb:(b + 1) * mb, :].astype(jnp.bfloat16)
            e = jnp.exp(xv - jnp.bfloat16(SHIFT))
            out_ref[b * mb:(b + 1) * mb, :] = e
            s_col = jnp.sum(e, axis=1, keepdims=True, dtype=jnp.float32)
            mine_ref[b, 0:1, :] = s_col.T

            if b == 0:
                pl.semaphore_wait(barrier_sem, N_DEV - 1)

            for p in range(N_DEV):
                @pl.when(my_pos != p)
                def _():
                    rdma = pltpu.make_async_remote_copy(
                        src_ref=mine_ref.at[b],
                        dst_ref=comm_ref.at[my_pos, b],
                        send_sem=send_sems.at[b, p],
                        recv_sem=recv_sems.at[b, my_pos],
                        device_id=(p,),
                        device_id_type=pl.DeviceIdType.MESH,
                    )
                    rdma.start()
            comm_ref[my_pos, b] = mine_ref[b]

        for b in range(N_BLK):
            for src in range(N_DEV):
                @pl.when(my_pos != src)
                def _():
                    recv = pltpu.make_async_remote_copy(
                        src_ref=mine_ref.at[b],
                        dst_ref=comm_ref.at[src, b],
                        send_sem=send_sems.at[b, src],
                        recv_sem=recv_sems.at[b, src],
                        device_id=(src,),
                        device_id_type=pl.DeviceIdType.MESH,
                    )
                    recv.wait_recv()

            denom_row = jnp.sum(comm_ref[:, b, 0, :], axis=0, keepdims=True)
            factor_col = (1.0 / denom_row).T
            out_ref[b * mb:(b + 1) * mb, :] = (
                out_ref[b * mb:(b + 1) * mb, :].astype(jnp.float32)
                * factor_col
            ).astype(jnp.bfloat16)

        for b in range(N_BLK):
            for p in range(N_DEV):
                @pl.when(my_pos != p)
                def _():
                    send = pltpu.make_async_remote_copy(
                        src_ref=mine_ref.at[b],
                        dst_ref=comm_ref.at[my_pos, b],
                        send_sem=send_sems.at[b, p],
                        recv_sem=recv_sems.at[b, my_pos],
                        device_id=(p,),
                        device_id_type=pl.DeviceIdType.MESH,
                    )
                    send.wait_send()

    return pl.pallas_call(
        body,
        out_shape=jax.ShapeDtypeStruct((m, n), jnp.bfloat16),
        in_specs=[pl.BlockSpec(memory_space=pltpu.VMEM)],
        out_specs=pl.BlockSpec(memory_space=pltpu.VMEM),
        scratch_shapes=[
            pltpu.VMEM((N_BLK, 1, mb), jnp.float32),
            pltpu.VMEM((N_DEV, N_BLK, 1, mb), jnp.float32),
            pltpu.SemaphoreType.DMA((N_BLK, N_DEV)),
            pltpu.SemaphoreType.DMA((N_BLK, N_DEV)),
        ],
        compiler_params=pltpu.CompilerParams(collective_id=0),
    )(x)


# device time: 10958 ns/iter; 1.0518x vs baseline; 1.0518x over previous
import jax
import jax.numpy as jnp
from jax import lax
from jax.experimental import pallas as pl
from jax.experimental.pallas import tpu as pltpu

N_DEV = 8
N_BLK = 4
SHIFT = 16.0


def kernel(x):
    m, n = x.shape
    mb = m // N_BLK

    def body(x_ref, out_ref, mine_ref, comm_ref, send_sems, recv_sems):
        my_pos = lax.axis_index("i")

        barrier_sem = pltpu.get_barrier_semaphore()
        for p in range(N_DEV):
            @pl.when(my_pos != p)
            def _():
                pl.semaphore_signal(
                    barrier_sem, inc=1,
                    device_id=(p,), device_id_type=pl.DeviceIdType.MESH,
                )

        for b in range(N_BLK):
            e32 = jnp.exp(x_ref[b * mb:(b + 1) * mb, :] - SHIFT)
            out_ref[b * mb:(b + 1) * mb, :] = e32.astype(jnp.bfloat16)
            s_col = jnp.sum(e32, axis=1, keepdims=True, dtype=jnp.float32)
            mine_ref[b, 0:1, :] = s_col.T

            if b == 0:
                pl.semaphore_wait(barrier_sem, N_DEV - 1)

            for p in range(N_DEV):
                @pl.when(my_pos != p)
                def _():
                    rdma = pltpu.make_async_remote_copy(
                        src_ref=mine_ref.at[b],
                        dst_ref=comm_ref.at[my_pos, b],
                        send_sem=send_sems.at[b, p],
                        recv_sem=recv_sems.at[b, my_pos],
                        device_id=(p,),
                        device_id_type=pl.DeviceIdType.MESH,
                    )
                    rdma.start()
            comm_ref[my_pos, b] = mine_ref[b]

        for b in range(N_BLK):
            for src in range(N_DEV):
                @pl.when(my_pos != src)
                def _():
                    recv = pltpu.make_async_remote_copy(
                        src_ref=mine_ref.at[b],
                        dst_ref=comm_ref.at[src, b],
                        send_sem=send_sems.at[b, src],
                        recv_sem=recv_sems.at[b, src],
                        device_id=(src,),
                        device_id_type=pl.DeviceIdType.MESH,
                    )
                    recv.wait_recv()

            denom_row = jnp.sum(comm_ref[:, b, 0, :], axis=0, keepdims=True)
            factor_col = (1.0 / denom_row).T
            out_ref[b * mb:(b + 1) * mb, :] = (
                out_ref[b * mb:(b + 1) * mb, :].astype(jnp.float32)
                * factor_col
            ).astype(jnp.bfloat16)

        for b in range(N_BLK):
            for p in range(N_DEV):
                @pl.when(my_pos != p)
                def _():
                    send = pltpu.make_async_remote_copy(
                        src_ref=mine_ref.at[b],
                        dst_ref=comm_ref.at[my_pos, b],
                        send_sem=send_sems.at[b, p],
                        recv_sem=recv_sems.at[b, my_pos],
                        device_id=(p,),
                        device_id_type=pl.DeviceIdType.MESH,
                    )
                    send.wait_send()

    return pl.pallas_call(
        body,
        out_shape=jax.ShapeDtypeStruct((m, n), jnp.bfloat16),
        in_specs=[pl.BlockSpec(memory_space=pltpu.VMEM)],
        out_specs=pl.BlockSpec(memory_space=pltpu.VMEM),
        scratch_shapes=[
            pltpu.VMEM((N_BLK, 1, mb), jnp.float32),
            pltpu.VMEM((N_DEV, N_BLK, 1, mb), jnp.float32),
            pltpu.SemaphoreType.DMA((N_BLK, N_DEV)),
            pltpu.SemaphoreType.DMA((N_BLK, N_DEV)),
        ],
        compiler_params=pltpu.CompilerParams(collective_id=0),
    )(x)


# device time: 10944 ns/iter; 1.0532x vs baseline; 1.0013x over previous
import jax
import jax.numpy as jnp
from jax import lax
from jax.experimental import pallas as pl
from jax.experimental.pallas import tpu as pltpu

N_DEV = 8
N_BLK = 2
SHIFT = 16.0


def kernel(x):
    m, n = x.shape
    mb = m // N_BLK

    def body(x_ref, out_ref, mine_ref, comm_ref, send_sems, recv_sems):
        my_pos = lax.axis_index("i")

        barrier_sem = pltpu.get_barrier_semaphore()
        for p in range(N_DEV):
            @pl.when(my_pos != p)
            def _():
                pl.semaphore_signal(
                    barrier_sem, inc=1,
                    device_id=(p,), device_id_type=pl.DeviceIdType.MESH,
                )

        for b in range(N_BLK):
            e32 = jnp.exp(x_ref[b * mb:(b + 1) * mb, :] - SHIFT)
            out_ref[b * mb:(b + 1) * mb, :] = e32.astype(jnp.bfloat16)
            s_col = jnp.sum(e32, axis=1, keepdims=True, dtype=jnp.float32)
            mine_ref[b, 0:1, :] = s_col.T

            if b == 0:
                pl.semaphore_wait(barrier_sem, N_DEV - 1)

            for p in range(N_DEV):
                @pl.when(my_pos != p)
                def _():
                    rdma = pltpu.make_async_remote_copy(
                        src_ref=mine_ref.at[b],
                        dst_ref=comm_ref.at[my_pos, b],
                        send_sem=send_sems.at[b, p],
                        recv_sem=recv_sems.at[b, my_pos],
                        device_id=(p,),
                        device_id_type=pl.DeviceIdType.MESH,
                    )
                    rdma.start()
            comm_ref[my_pos, b] = mine_ref[b]

        for b in range(N_BLK):
            for src in range(N_DEV):
                @pl.when(my_pos != src)
                def _():
                    recv = pltpu.make_async_remote_copy(
                        src_ref=mine_ref.at[b],
                        dst_ref=comm_ref.at[src, b],
                        send_sem=send_sems.at[b, src],
                        recv_sem=recv_sems.at[b, src],
                        device_id=(src,),
                        device_id_type=pl.DeviceIdType.MESH,
                    )
                    recv.wait_recv()

            denom_row = jnp.sum(comm_ref[:, b, 0, :], axis=0, keepdims=True)
            factor_col = (1.0 / denom_row).T
            out_ref[b * mb:(b + 1) * mb, :] = (
                out_ref[b * mb:(b + 1) * mb, :].astype(jnp.float32)
                * factor_col
            ).astype(jnp.bfloat16)

        for b in range(N_BLK):
            for p in range(N_DEV):
                @pl.when(my_pos != p)
                def _():
                    send = pltpu.make_async_remote_copy(
                        src_ref=mine_ref.at[b],
                        dst_ref=comm_ref.at[my_pos, b],
                        send_sem=send_sems.at[b, p],
                        recv_sem=recv_sems.at[b, my_pos],
                        device_id=(p,),
                        device_id_type=pl.DeviceIdType.MESH,
                    )
                    send.wait_send()

    return pl.pallas_call(
        body,
        out_shape=jax.ShapeDtypeStruct((m, n), jnp.bfloat16),
        in_specs=[pl.BlockSpec(memory_space=pltpu.VMEM)],
        out_specs=pl.BlockSpec(memory_space=pltpu.VMEM),
        scratch_shapes=[
            pltpu.VMEM((N_BLK, 1, mb), jnp.float32),
            pltpu.VMEM((N_DEV, N_BLK, 1, mb), jnp.float32),
            pltpu.SemaphoreType.DMA((N_BLK, N_DEV)),
            pltpu.SemaphoreType.DMA((N_BLK, N_DEV)),
        ],
        compiler_params=pltpu.CompilerParams(collective_id=0),
    )(x)
